# baseline (speedup 1.0000x reference)
import functools

import numpy as np
import jax
import jax.numpy as jnp
from jax import lax
from jax.experimental import pallas as pl
from jax.experimental.pallas import tpu as pltpu
from jax.experimental.pallas import tpu_sc as plsc

_CAPACITY = 0.125
_EPS = 1e-6
_HD = 128


def _router_body(hs_ref, w_ref, gidx_ref, gate_ref, lg_scr, *, k, t_chunk):
    b = pl.program_id(0)
    c = pl.program_id(1)
    nc = pl.num_programs(1)
    t_total = lg_scr.shape[1]

    x = hs_ref[0]
    lg = lax.dot_general(w_ref[...], x, (((1,), (1,)), ((), ())),
                         preferred_element_type=jnp.float32)
    lg_scr[:, pl.ds(c * t_chunk, t_chunk)] = lg

    @pl.when(c == nc - 1)
    def _():
        lg_all = lg_scr[...]
        ibits = lax.bitcast_convert_type(lg_all, jnp.int32)
        skey = jnp.where(ibits >= 0, ibits, ibits ^ jnp.int32(0x7FFFFFFF))

        def search_step(_, carry):
            lo, hi = carry
            xo = lo ^ hi
            mid = (lo & hi) + (xo >> 1) + (xo & 1)
            cnt = jnp.sum((skey >= mid).astype(jnp.int32))
            ok = cnt >= k
            return jnp.where(ok, mid, lo), jnp.where(ok, hi, mid - 1)

        v, _ = lax.fori_loop(
            0, 32, search_step,
            (jnp.int32(-2**31), jnp.int32(2**31 - 1)))
        cnt_gt = jnp.sum((skey > v).astype(jnp.int32))
        need = k - cnt_gt
        eq = skey == v

        def icumsum(m):
            cs = m
            s = 1
            while s < t_total:
                z = jnp.zeros((1, s), jnp.int32)
                cs = cs + jnp.concatenate([z, cs[:, :t_total - s]], axis=1)
                s *= 2
            return cs

        eq_cs = icumsum(eq.astype(jnp.int32))
        mask = (skey > v) | (eq & (eq_cs <= need))
        pos = icumsum(mask.astype(jnp.int32)) - 1
        tvec = lax.broadcasted_iota(jnp.int32, (1, t_total), 1)
        jc_blk = 128
        for jc in range(k // jc_blk):
            jmat = lax.broadcasted_iota(jnp.int32, (jc_blk, t_total), 0) + jc * jc_blk
            em = (pos == jmat) & mask
            tok = jnp.sum(jnp.where(em, tvec, 0), axis=1, keepdims=True)
            sc = jnp.sum(jnp.where(em, lg_all, 0.0), axis=1, keepdims=True)
            gidx_ref[0, pl.ds(jc * jc_blk, jc_blk), :] = tok + b * t_total
            gate_ref[0, pl.ds(jc * jc_blk, jc_blk), :] = 1.0 / (1.0 + jnp.exp(-sc))


def _route(hidden_states, w_router, k):
    B, T, D = hidden_states.shape
    ct = 512
    gidx, gate = pl.pallas_call(
        functools.partial(_router_body, k=k, t_chunk=ct),
        grid=(B, T // ct),
        in_specs=[
            pl.BlockSpec((1, ct, D), lambda b, c: (b, c, 0)),
            pl.BlockSpec((1, D), lambda b, c: (0, 0)),
        ],
        out_specs=[
            pl.BlockSpec((1, k, 1), lambda b, c: (b, 0, 0)),
            pl.BlockSpec((1, k, 1), lambda b, c: (b, 0, 0)),
        ],
        out_shape=[
            jax.ShapeDtypeStruct((B, k, 1), jnp.int32),
            jax.ShapeDtypeStruct((B, k, 1), jnp.float32),
        ],
        scratch_shapes=[pltpu.VMEM((1, T), jnp.float32)],
    )(hidden_states, w_router.reshape(1, D))
    return gidx, gate


def _gather(hs2d, gidx_flat):
    n = gidx_flat.shape[0]
    d = hs2d.shape[1]
    info = plsc.get_sparse_core_info()
    nw = info.num_cores * info.num_subcores
    rpw = n // nw
    rch = min(rpw, 32)
    mesh = plsc.VectorSubcoreMesh(core_axis_name="c", subcore_axis_name="s")

    @functools.partial(
        pl.kernel,
        out_type=jax.ShapeDtypeStruct((n, d), jnp.float32),
        mesh=mesh,
        scratch_types=[
            pltpu.VMEM((rch,), jnp.int32),
            pltpu.VMEM((rch, d), jnp.float32),
            pltpu.SemaphoreType.DMA,
        ],
    )
    def gather_kernel(hs_ref, gidx_ref, out_ref, idx_v, rows_v, sem):
        wid = lax.axis_index("s") * info.num_cores + lax.axis_index("c")
        base = wid * rpw
        for h in range(rpw // rch):
            off = base + h * rch
            pltpu.sync_copy(gidx_ref.at[pl.ds(off, rch)], idx_v)
            pltpu.async_copy(hs_ref.at[idx_v], rows_v, sem).wait()
            pltpu.sync_copy(rows_v, out_ref.at[pl.ds(off, rch)])

    return gather_kernel(hs2d, gidx_flat)


def _qkv_body(x_ref, wq_ref, wk_ref, wv_ref, bq_ref, bk_ref, bv_ref,
              cos_ref, sin_ref, ln1_ref, q_ref, k_ref, v_ref):
    x = x_ref[...]
    var = jnp.mean(x * x, axis=-1, keepdims=True)
    h = x * lax.rsqrt(var + _EPS) * ln1_ref[...]
    q = jnp.dot(h, wq_ref[...], preferred_element_type=jnp.float32) + bq_ref[...]
    kk = jnp.dot(h, wk_ref[...], preferred_element_type=jnp.float32) + bk_ref[...]
    v = jnp.dot(h, wv_ref[...], preferred_element_type=jnp.float32) + bv_ref[...]
    cos = cos_ref[...]
    sin = sin_ref[...]

    def rope(z):
        outs = []
        for hh in range(z.shape[1] // _HD):
            zz = z[:, hh * _HD:(hh + 1) * _HD]
            z1 = zz[:, :_HD // 2]
            z2 = zz[:, _HD // 2:]
            rot = jnp.concatenate([-z2, z1], axis=1)
            outs.append(zz * cos + rot * sin)
        return jnp.concatenate(outs, axis=1)

    q_ref[...] = rope(q)
    k_ref[...] = rope(kk)
    v_ref[...] = v


def _qkv(sel, wq, bq, wk, bk, wv, bv, ln1_w, cos, sin):
    n, d = sel.shape
    hhd = wq.shape[1]
    rb, cb = 512, 512
    outs = pl.pallas_call(
        _qkv_body,
        grid=(n // rb, hhd // cb),
        in_specs=[
            pl.BlockSpec((rb, d), lambda r, c: (r, 0)),
            pl.BlockSpec((d, cb), lambda r, c: (0, c)),
            pl.BlockSpec((d, cb), lambda r, c: (0, c)),
            pl.BlockSpec((d, cb), lambda r, c: (0, c)),
            pl.BlockSpec((1, cb), lambda r, c: (0, c)),
            pl.BlockSpec((1, cb), lambda r, c: (0, c)),
            pl.BlockSpec((1, cb), lambda r, c: (0, c)),
            pl.BlockSpec((rb, _HD), lambda r, c: (r, 0)),
            pl.BlockSpec((rb, _HD), lambda r, c: (r, 0)),
            pl.BlockSpec((1, d), lambda r, c: (0, 0)),
        ],
        out_specs=[pl.BlockSpec((rb, cb), lambda r, c: (r, c))] * 3,
        out_shape=[jax.ShapeDtypeStruct((n, hhd), jnp.float32)] * 3,
    )(sel, wq, wk, wv, bq.reshape(1, hhd), bk.reshape(1, hhd),
      bv.reshape(1, hhd), cos, sin, ln1_w.reshape(1, d))
    return outs


def _attn_body(q_ref, k_ref, v_ref, o_ref, *, bq_rows, scale):
    qi = pl.program_id(1)
    q = q_ref[...]
    s = lax.dot_general(q, k_ref[...], (((1,), (1,)), ((), ())),
                        preferred_element_type=jnp.float32) * scale
    rows = lax.broadcasted_iota(jnp.int32, s.shape, 0) + qi * bq_rows
    cols = lax.broadcasted_iota(jnp.int32, s.shape, 1)
    s = jnp.where(cols <= rows, s, jnp.float32(-1e9))
    m = jnp.max(s, axis=-1, keepdims=True)
    e = jnp.exp(s - m)
    p = e / jnp.sum(e, axis=-1, keepdims=True)
    o_ref[...] = jnp.dot(p, v_ref[...], preferred_element_type=jnp.float32)


def _attn(q, k, v):
    n, hhd = q.shape
    nh = hhd // _HD
    bq = 512
    return pl.pallas_call(
        functools.partial(_attn_body, bq_rows=bq, scale=1.0 / float(np.sqrt(_HD))),
        grid=(nh, n // bq),
        in_specs=[
            pl.BlockSpec((bq, _HD), lambda h, qi: (qi, h)),
            pl.BlockSpec((n, _HD), lambda h, qi: (0, h)),
            pl.BlockSpec((n, _HD), lambda h, qi: (0, h)),
        ],
        out_specs=pl.BlockSpec((bq, _HD), lambda h, qi: (qi, h)),
        out_shape=jax.ShapeDtypeStruct((n, hhd), jnp.float32),
    )(q, k, v)


def _oproj_body(att_ref, wo_ref, sel_ref, ln2_ref, x1_ref, h2_ref):
    x1 = jnp.dot(att_ref[...], wo_ref[...],
                 preferred_element_type=jnp.float32) + sel_ref[...]
    x1_ref[...] = x1
    var = jnp.mean(x1 * x1, axis=-1, keepdims=True)
    h2_ref[...] = x1 * lax.rsqrt(var + _EPS) * ln2_ref[...]


def _oproj(att, wo, sel, ln2_w):
    n, d = sel.shape
    rb = 512
    return pl.pallas_call(
        _oproj_body,
        grid=(n // rb,),
        in_specs=[
            pl.BlockSpec((rb, wo.shape[0]), lambda r: (r, 0)),
            pl.BlockSpec(wo.shape, lambda r: (0, 0)),
            pl.BlockSpec((rb, d), lambda r: (r, 0)),
            pl.BlockSpec((1, d), lambda r: (0, 0)),
        ],
        out_specs=[pl.BlockSpec((rb, d), lambda r: (r, 0))] * 2,
        out_shape=[jax.ShapeDtypeStruct((n, d), jnp.float32)] * 2,
    )(att, wo, sel, ln2_w.reshape(1, d))


def _mlp_body(h2_ref, wg_ref, wu_ref, wd_ref, x1_ref, sel_ref, gate_ref, out_ref):
    i = pl.program_id(1)
    ni = pl.num_programs(1)
    h2 = h2_ref[...]
    g = jnp.dot(h2, wg_ref[...], preferred_element_type=jnp.float32)
    u = jnp.dot(h2, wu_ref[...], preferred_element_type=jnp.float32)
    m = g * (1.0 / (1.0 + jnp.exp(-g))) * u
    part = jnp.dot(m, wd_ref[...], preferred_element_type=jnp.float32)

    @pl.when(i == 0)
    def _():
        out_ref[...] = part

    @pl.when(i > 0)
    def _():
        out_ref[...] += part

    @pl.when(i == ni - 1)
    def _():
        y = out_ref[...] + x1_ref[...]
        gt = gate_ref[...]
        out_ref[...] = gt * y + (1.0 - gt) * sel_ref[...]


def _mlp(h2, w_gate, w_up, w_down, x1, sel, gate_col):
    n, d = h2.shape
    i_dim = w_gate.shape[1]
    rb, ci = 512, 256
    return pl.pallas_call(
        _mlp_body,
        grid=(n // rb, i_dim // ci),
        in_specs=[
            pl.BlockSpec((rb, d), lambda r, i: (r, 0)),
            pl.BlockSpec((d, ci), lambda r, i: (0, i)),
            pl.BlockSpec((d, ci), lambda r, i: (0, i)),
            pl.BlockSpec((ci, d), lambda r, i: (i, 0)),
            pl.BlockSpec((rb, d), lambda r, i: (r, 0)),
            pl.BlockSpec((rb, d), lambda r, i: (r, 0)),
            pl.BlockSpec((rb, 1), lambda r, i: (r, 0)),
        ],
        out_specs=pl.BlockSpec((rb, d), lambda r, i: (r, 0)),
        out_shape=jax.ShapeDtypeStruct((n, d), jnp.float32),
    )(h2, w_gate, w_up, w_down, x1, sel, gate_col)


def _scatter_body(gidx_sref, gated_ref, hs_any, out_ref):
    del gidx_sref, hs_any
    out_ref[...] = gated_ref[...]


def _scatter(gidx_flat, gated, hs2d):
    n, d = gated.shape
    bt = hs2d.shape[0]
    grid_spec = pltpu.PrefetchScalarGridSpec(
        num_scalar_prefetch=1,
        grid=(n,),
        in_specs=[
            pl.BlockSpec((1, 1, d), lambda nn, g: (nn, 0, 0)),
            pl.BlockSpec(memory_space=pl.ANY),
        ],
        out_specs=pl.BlockSpec((1, 1, d), lambda nn, g: (g[nn], 0, 0)),
    )
    out3 = pl.pallas_call(
        _scatter_body,
        grid_spec=grid_spec,
        out_shape=jax.ShapeDtypeStruct((bt, 1, d), jnp.float32),
        input_output_aliases={2: 0},
    )(gidx_flat, gated.reshape(n, 1, d), hs2d.reshape(bt, 1, d))
    return out3.reshape(bt, d)


def kernel(hidden_states, w_router, wq, bq, wk, bk, wv, bv, wo,
           w_gate, w_up, w_down, ln1_w, ln2_w):
    B, T, D = hidden_states.shape
    k = max(1, int(T * _CAPACITY))
    n = B * k

    gidx, gate = _route(hidden_states, w_router, k)
    gidx_flat = gidx.reshape(n)
    gate_col = gate.reshape(n, 1)
    hs2d = hidden_states.reshape(B * T, D)

    sel = _gather(hs2d, gidx_flat)

    pos = jnp.arange(n, dtype=jnp.float32)
    inv = 1.0 / (10000.0 ** (jnp.arange(0, _HD, 2, dtype=jnp.float32) / _HD))
    ang = pos[:, None] * inv[None, :]
    cos = jnp.concatenate([jnp.cos(ang), jnp.cos(ang)], axis=-1)
    sin = jnp.concatenate([jnp.sin(ang), jnp.sin(ang)], axis=-1)

    q, kk, v = _qkv(sel, wq, bq, wk, bk, wv, bv, ln1_w, cos, sin)
    att = _attn(q, kk, v)
    x1, h2 = _oproj(att, wo, sel, ln2_w)
    gated = _mlp(h2, w_gate, w_up, w_down, x1, sel, gate_col)

    out2d = _scatter(gidx_flat, gated, hs2d)
    return out2d.reshape(B, T, D)


# speedup vs baseline: 2.7025x; 2.7025x over previous
import functools

import numpy as np
import jax
import jax.numpy as jnp
from jax import lax
from jax.experimental import pallas as pl
from jax.experimental.pallas import tpu as pltpu
from jax.experimental.pallas import tpu_sc as plsc

_CAPACITY = 0.125
_EPS = 1e-6
_HD = 128


def _router_body(hs_ref, w_ref, gidx_ref, gate_ref, ends_ref, lg_scr, *, k, t_chunk):
    b = pl.program_id(0)
    c = pl.program_id(1)
    nc = pl.num_programs(1)
    t_total = lg_scr.shape[1]

    x = hs_ref[0]
    lg = lax.dot_general(w_ref[...], x, (((1,), (1,)), ((), ())),
                         preferred_element_type=jnp.float32)
    lg_scr[:, pl.ds(c * t_chunk, t_chunk)] = lg

    @pl.when(c == nc - 1)
    def _():
        lg_all = lg_scr[...]
        ibits = lax.bitcast_convert_type(lg_all, jnp.int32)
        skey = jnp.where(ibits >= 0, ibits, ibits ^ jnp.int32(0x7FFFFFFF))

        def search_step(_, carry):
            lo, hi = carry
            xo = lo ^ hi
            mid = (lo & hi) + (xo >> 1) + (xo & 1)
            cnt = jnp.sum((skey >= mid).astype(jnp.int32))
            ok = cnt >= k
            return jnp.where(ok, mid, lo), jnp.where(ok, hi, mid - 1)

        v, _ = lax.fori_loop(
            0, 32, search_step,
            (jnp.int32(-2**31), jnp.int32(2**31 - 1)))
        cnt_gt = jnp.sum((skey > v).astype(jnp.int32))
        need = k - cnt_gt
        eq = skey == v

        def icumsum(m):
            cs = m
            s = 1
            while s < t_total:
                z = jnp.zeros((1, s), jnp.int32)
                cs = cs + jnp.concatenate([z, cs[:, :t_total - s]], axis=1)
                s *= 2
            return cs

        eq_cs = icumsum(eq.astype(jnp.int32))
        mask = (skey > v) | (eq & (eq_cs <= need))
        pos = icumsum(mask.astype(jnp.int32)) - 1
        tvec = lax.broadcasted_iota(jnp.int32, (1, t_total), 1)
        nb = t_total // 512
        bmat = lax.broadcasted_iota(jnp.int32, (nb, t_total), 0)
        in_blk = tvec < (bmat + 1) * 512
        ends_ref[0] = jnp.sum(jnp.where(in_blk & mask, 1, 0), axis=1,
                              keepdims=True)
        jc_blk = 128
        for jc in range(k // jc_blk):
            jmat = lax.broadcasted_iota(jnp.int32, (jc_blk, t_total), 0) + jc * jc_blk
            em = (pos == jmat) & mask
            tok = jnp.sum(jnp.where(em, tvec, 0), axis=1, keepdims=True)
            sc = jnp.sum(jnp.where(em, lg_all, 0.0), axis=1, keepdims=True)
            gidx_ref[0, pl.ds(jc * jc_blk, jc_blk), :] = tok + b * t_total
            gate_ref[0, pl.ds(jc * jc_blk, jc_blk), :] = 1.0 / (1.0 + jnp.exp(-sc))


def _route(hidden_states, w_router, k):
    B, T, D = hidden_states.shape
    ct = 512
    nb = T // 512
    gidx, gate, ends = pl.pallas_call(
        functools.partial(_router_body, k=k, t_chunk=ct),
        grid=(B, T // ct),
        in_specs=[
            pl.BlockSpec((1, ct, D), lambda b, c: (b, c, 0)),
            pl.BlockSpec((1, D), lambda b, c: (0, 0)),
        ],
        out_specs=[
            pl.BlockSpec((1, k, 1), lambda b, c: (b, 0, 0)),
            pl.BlockSpec((1, k, 1), lambda b, c: (b, 0, 0)),
            pl.BlockSpec((1, nb, 1), lambda b, c: (b, 0, 0)),
        ],
        out_shape=[
            jax.ShapeDtypeStruct((B, k, 1), jnp.int32),
            jax.ShapeDtypeStruct((B, k, 1), jnp.float32),
            jax.ShapeDtypeStruct((B, nb, 1), jnp.int32),
        ],
        scratch_shapes=[pltpu.VMEM((1, T), jnp.float32)],
    )(hidden_states, w_router.reshape(1, D))
    return gidx, gate, ends


def _gather(hs2d, gidx_flat):
    n = gidx_flat.shape[0]
    d = hs2d.shape[1]
    info = plsc.get_sparse_core_info()
    nw = info.num_cores * info.num_subcores
    rpw = n // nw
    rch = min(rpw, 32)
    mesh = plsc.VectorSubcoreMesh(core_axis_name="c", subcore_axis_name="s")

    @functools.partial(
        pl.kernel,
        out_type=jax.ShapeDtypeStruct((n, d), jnp.float32),
        mesh=mesh,
        scratch_types=[
            pltpu.VMEM((rch,), jnp.int32),
            pltpu.VMEM((rch, d), jnp.float32),
            pltpu.SemaphoreType.DMA,
        ],
    )
    def gather_kernel(hs_ref, gidx_ref, out_ref, idx_v, rows_v, sem):
        wid = lax.axis_index("s") * info.num_cores + lax.axis_index("c")
        base = wid * rpw
        for h in range(rpw // rch):
            off = base + h * rch
            pltpu.sync_copy(gidx_ref.at[pl.ds(off, rch)], idx_v)
            pltpu.async_copy(hs_ref.at[idx_v], rows_v, sem).wait()
            pltpu.sync_copy(rows_v, out_ref.at[pl.ds(off, rch)])

    return gather_kernel(hs2d, gidx_flat)


def _qkv_body(x_ref, wq_ref, wk_ref, wv_ref, bq_ref, bk_ref, bv_ref,
              cos_ref, sin_ref, ln1_ref, q_ref, k_ref, v_ref):
    x = x_ref[...]
    var = jnp.mean(x * x, axis=-1, keepdims=True)
    h = (x * lax.rsqrt(var + _EPS) * ln1_ref[...]).astype(jnp.bfloat16)
    wq = wq_ref[...].astype(jnp.bfloat16)
    wk = wk_ref[...].astype(jnp.bfloat16)
    wv = wv_ref[...].astype(jnp.bfloat16)
    q = jnp.dot(h, wq, preferred_element_type=jnp.float32) + bq_ref[...]
    kk = jnp.dot(h, wk, preferred_element_type=jnp.float32) + bk_ref[...]
    v = jnp.dot(h, wv, preferred_element_type=jnp.float32) + bv_ref[...]
    cos = cos_ref[...]
    sin = sin_ref[...]

    def rope(z):
        outs = []
        for hh in range(z.shape[1] // _HD):
            zz = z[:, hh * _HD:(hh + 1) * _HD]
            z1 = zz[:, :_HD // 2]
            z2 = zz[:, _HD // 2:]
            rot = jnp.concatenate([-z2, z1], axis=1)
            outs.append(zz * cos + rot * sin)
        return jnp.concatenate(outs, axis=1)

    q_ref[...] = rope(q)
    k_ref[...] = rope(kk)
    v_ref[...] = v


def _qkv(sel, wq, bq, wk, bk, wv, bv, ln1_w, cos, sin):
    n, d = sel.shape
    hhd = wq.shape[1]
    rb, cb = 512, 512
    outs = pl.pallas_call(
        _qkv_body,
        grid=(n // rb, hhd // cb),
        in_specs=[
            pl.BlockSpec((rb, d), lambda r, c: (r, 0)),
            pl.BlockSpec((d, cb), lambda r, c: (0, c)),
            pl.BlockSpec((d, cb), lambda r, c: (0, c)),
            pl.BlockSpec((d, cb), lambda r, c: (0, c)),
            pl.BlockSpec((1, cb), lambda r, c: (0, c)),
            pl.BlockSpec((1, cb), lambda r, c: (0, c)),
            pl.BlockSpec((1, cb), lambda r, c: (0, c)),
            pl.BlockSpec((rb, _HD), lambda r, c: (r, 0)),
            pl.BlockSpec((rb, _HD), lambda r, c: (r, 0)),
            pl.BlockSpec((1, d), lambda r, c: (0, 0)),
        ],
        out_specs=[pl.BlockSpec((rb, cb), lambda r, c: (r, c))] * 3,
        out_shape=[jax.ShapeDtypeStruct((n, hhd), jnp.float32)] * 3,
    )(sel, wq, wk, wv, bq.reshape(1, hhd), bk.reshape(1, hhd),
      bv.reshape(1, hhd), cos, sin, ln1_w.reshape(1, d))
    return outs


def _attn_body(q_ref, k_ref, v_ref, o_ref, *, bq_rows, scale):
    qi = pl.program_id(1)
    q = q_ref[...].astype(jnp.bfloat16)
    kb = k_ref[...].astype(jnp.bfloat16)
    s = lax.dot_general(q, kb, (((1,), (1,)), ((), ())),
                        preferred_element_type=jnp.float32) * scale
    rows = lax.broadcasted_iota(jnp.int32, s.shape, 0) + qi * bq_rows
    cols = lax.broadcasted_iota(jnp.int32, s.shape, 1)
    s = jnp.where(cols <= rows, s, jnp.float32(-1e9))
    m = jnp.max(s, axis=-1, keepdims=True)
    e = jnp.exp(s - m)
    p = (e / jnp.sum(e, axis=-1, keepdims=True)).astype(jnp.bfloat16)
    vb = v_ref[...].astype(jnp.bfloat16)
    o_ref[...] = jnp.dot(p, vb, preferred_element_type=jnp.float32)


def _attn(q, k, v):
    n, hhd = q.shape
    nh = hhd // _HD
    bq = 512
    return pl.pallas_call(
        functools.partial(_attn_body, bq_rows=bq, scale=1.0 / float(np.sqrt(_HD))),
        grid=(nh, n // bq),
        in_specs=[
            pl.BlockSpec((bq, _HD), lambda h, qi: (qi, h)),
            pl.BlockSpec((n, _HD), lambda h, qi: (0, h)),
            pl.BlockSpec((n, _HD), lambda h, qi: (0, h)),
        ],
        out_specs=pl.BlockSpec((bq, _HD), lambda h, qi: (qi, h)),
        out_shape=jax.ShapeDtypeStruct((n, hhd), jnp.float32),
    )(q, k, v)


def _oproj_body(att_ref, wo_ref, sel_ref, ln2_ref, x1_ref, h2_ref):
    x1 = jnp.dot(att_ref[...].astype(jnp.bfloat16),
                 wo_ref[...].astype(jnp.bfloat16),
                 preferred_element_type=jnp.float32) + sel_ref[...]
    x1_ref[...] = x1
    var = jnp.mean(x1 * x1, axis=-1, keepdims=True)
    h2_ref[...] = x1 * lax.rsqrt(var + _EPS) * ln2_ref[...]


def _oproj(att, wo, sel, ln2_w):
    n, d = sel.shape
    rb = 512
    return pl.pallas_call(
        _oproj_body,
        grid=(n // rb,),
        in_specs=[
            pl.BlockSpec((rb, wo.shape[0]), lambda r: (r, 0)),
            pl.BlockSpec(wo.shape, lambda r: (0, 0)),
            pl.BlockSpec((rb, d), lambda r: (r, 0)),
            pl.BlockSpec((1, d), lambda r: (0, 0)),
        ],
        out_specs=[pl.BlockSpec((rb, d), lambda r: (r, 0))] * 2,
        out_shape=[jax.ShapeDtypeStruct((n, d), jnp.float32)] * 2,
    )(att, wo, sel, ln2_w.reshape(1, d))


def _mlp_body(h2_ref, wg_ref, wu_ref, wd_ref, x1_ref, sel_ref, gate_ref, out_ref):
    i = pl.program_id(1)
    ni = pl.num_programs(1)
    h2 = h2_ref[...].astype(jnp.bfloat16)
    g = jnp.dot(h2, wg_ref[...].astype(jnp.bfloat16),
                preferred_element_type=jnp.float32)
    u = jnp.dot(h2, wu_ref[...].astype(jnp.bfloat16),
                preferred_element_type=jnp.float32)
    m = (g * (1.0 / (1.0 + jnp.exp(-g))) * u).astype(jnp.bfloat16)
    part = jnp.dot(m, wd_ref[...].astype(jnp.bfloat16),
                   preferred_element_type=jnp.float32)

    @pl.when(i == 0)
    def _():
        out_ref[...] = part

    @pl.when(i > 0)
    def _():
        out_ref[...] += part

    @pl.when(i == ni - 1)
    def _():
        y = out_ref[...] + x1_ref[...]
        gt = gate_ref[...]
        out_ref[...] = gt * y + (1.0 - gt) * sel_ref[...]


def _mlp(h2, w_gate, w_up, w_down, x1, sel, gate_col):
    n, d = h2.shape
    i_dim = w_gate.shape[1]
    rb, ci = 512, 256
    return pl.pallas_call(
        _mlp_body,
        grid=(n // rb, i_dim // ci),
        in_specs=[
            pl.BlockSpec((rb, d), lambda r, i: (r, 0)),
            pl.BlockSpec((d, ci), lambda r, i: (0, i)),
            pl.BlockSpec((d, ci), lambda r, i: (0, i)),
            pl.BlockSpec((ci, d), lambda r, i: (i, 0)),
            pl.BlockSpec((rb, d), lambda r, i: (r, 0)),
            pl.BlockSpec((rb, d), lambda r, i: (r, 0)),
            pl.BlockSpec((rb, 1), lambda r, i: (r, 0)),
        ],
        out_specs=pl.BlockSpec((rb, d), lambda r, i: (r, 0)),
        out_shape=jax.ShapeDtypeStruct((n, d), jnp.float32),
    )(h2, w_gate, w_up, w_down, x1, sel, gate_col)


def _merge_body(starts_sref, gidx_sref, hs_ref, gated_ref, out_ref, *, rb):
    b = pl.program_id(0)
    j = pl.program_id(1)
    nj = pl.num_programs(1)
    out_ref[...] = hs_ref[...]
    base = b * (nj * rb) + j * rb

    def patch(n, carry):
        local = gidx_sref[n] - base
        out_ref[0, pl.ds(local, 1), :] = gated_ref[pl.ds(n, 1), :]
        return carry

    m = b * nj + j
    lax.fori_loop(starts_sref[m], starts_sref[m + 1], patch, 0)


def _merge(starts, gidx_flat, hidden_states, gated):
    B, T, D = hidden_states.shape
    n = gated.shape[0]
    rb = 512
    grid_spec = pltpu.PrefetchScalarGridSpec(
        num_scalar_prefetch=2,
        grid=(B, T // rb),
        in_specs=[
            pl.BlockSpec((1, rb, D), lambda b, j, s, g: (b, j, 0)),
            pl.BlockSpec((n, D), lambda b, j, s, g: (0, 0)),
        ],
        out_specs=pl.BlockSpec((1, rb, D), lambda b, j, s, g: (b, j, 0)),
    )
    return pl.pallas_call(
        functools.partial(_merge_body, rb=rb),
        grid_spec=grid_spec,
        out_shape=jax.ShapeDtypeStruct((B, T, D), jnp.float32),
    )(starts, gidx_flat, hidden_states, gated)


def kernel(hidden_states, w_router, wq, bq, wk, bk, wv, bv, wo,
           w_gate, w_up, w_down, ln1_w, ln2_w):
    B, T, D = hidden_states.shape
    k = max(1, int(T * _CAPACITY))
    n = B * k

    gidx, gate, ends = _route(hidden_states, w_router, k)
    gidx_flat = gidx.reshape(n)
    gate_col = gate.reshape(n, 1)
    hs2d = hidden_states.reshape(B * T, D)
    nb = T // 512
    ends_flat = ends.reshape(B, nb) + (jnp.arange(B, dtype=jnp.int32) * k)[:, None]
    starts = jnp.concatenate(
        [jnp.zeros((1,), jnp.int32), ends_flat.reshape(B * nb)])

    sel = _gather(hs2d, gidx_flat)

    pos = jnp.arange(n, dtype=jnp.float32)
    inv = 1.0 / (10000.0 ** (jnp.arange(0, _HD, 2, dtype=jnp.float32) / _HD))
    ang = pos[:, None] * inv[None, :]
    cos = jnp.concatenate([jnp.cos(ang), jnp.cos(ang)], axis=-1)
    sin = jnp.concatenate([jnp.sin(ang), jnp.sin(ang)], axis=-1)

    q, kk, v = _qkv(sel, wq, bq, wk, bk, wv, bv, ln1_w, cos, sin)
    att = _attn(q, kk, v)
    x1, h2 = _oproj(att, wo, sel, ln2_w)
    gated = _mlp(h2, w_gate, w_up, w_down, x1, sel, gate_col)

    return _merge(starts, gidx_flat, hidden_states, gated)


# speedup vs baseline: 3.1771x; 1.1756x over previous
import functools

import numpy as np
import jax
import jax.numpy as jnp
from jax import lax
from jax.experimental import pallas as pl
from jax.experimental.pallas import tpu as pltpu
from jax.experimental.pallas import tpu_sc as plsc

_CAPACITY = 0.125
_EPS = 1e-6
_HD = 128


def _router_body(hs_ref, w_ref, gidx_ref, gate_ref, ends_ref, lg_scr, *, k, t_chunk):
    b = pl.program_id(0)
    c = pl.program_id(1)
    nc = pl.num_programs(1)
    t_total = lg_scr.shape[1]

    x = hs_ref[0]
    lg = lax.dot_general(w_ref[...], x, (((1,), (1,)), ((), ())),
                         preferred_element_type=jnp.float32)
    lg_scr[:, pl.ds(c * t_chunk, t_chunk)] = lg

    @pl.when(c == nc - 1)
    def _():
        lg_all = lg_scr[...]
        ibits = lax.bitcast_convert_type(lg_all, jnp.int32)
        skey = jnp.where(ibits >= 0, ibits, ibits ^ jnp.int32(0x7FFFFFFF))

        def search_step(_, carry):
            lo, hi = carry
            xo = lo ^ hi
            mid = (lo & hi) + (xo >> 1) + (xo & 1)
            cnt = jnp.sum((skey >= mid).astype(jnp.int32))
            ok = cnt >= k
            return jnp.where(ok, mid, lo), jnp.where(ok, hi, mid - 1)

        v, _ = lax.fori_loop(
            0, 32, search_step,
            (jnp.int32(-2**31), jnp.int32(2**31 - 1)))
        cnt_gt = jnp.sum((skey > v).astype(jnp.int32))
        need = k - cnt_gt
        eq = skey == v

        def icumsum(m):
            cs = m
            s = 1
            while s < t_total:
                z = jnp.zeros((1, s), jnp.int32)
                cs = cs + jnp.concatenate([z, cs[:, :t_total - s]], axis=1)
                s *= 2
            return cs

        eq_cs = icumsum(eq.astype(jnp.int32))
        mask = (skey > v) | (eq & (eq_cs <= need))
        pos = icumsum(mask.astype(jnp.int32)) - 1
        tvec = lax.broadcasted_iota(jnp.int32, (1, t_total), 1)
        nb = t_total // 512
        bmat = lax.broadcasted_iota(jnp.int32, (nb, t_total), 0)
        in_blk = tvec < (bmat + 1) * 512
        ends_ref[0] = jnp.sum(jnp.where(in_blk & mask, 1, 0), axis=1,
                              keepdims=True)
        jc_blk = 128
        for jc in range(k // jc_blk):
            jmat = lax.broadcasted_iota(jnp.int32, (jc_blk, t_total), 0) + jc * jc_blk
            em = (pos == jmat) & mask
            tok = jnp.sum(jnp.where(em, tvec, 0), axis=1, keepdims=True)
            sc = jnp.sum(jnp.where(em, lg_all, 0.0), axis=1, keepdims=True)
            gidx_ref[0, pl.ds(jc * jc_blk, jc_blk), :] = tok + b * t_total
            gate_ref[0, pl.ds(jc * jc_blk, jc_blk), :] = 1.0 / (1.0 + jnp.exp(-sc))


def _route(hidden_states, w_router, k):
    B, T, D = hidden_states.shape
    ct = 512
    nb = T // 512
    gidx, gate, ends = pl.pallas_call(
        functools.partial(_router_body, k=k, t_chunk=ct),
        grid=(B, T // ct),
        in_specs=[
            pl.BlockSpec((1, ct, D), lambda b, c: (b, c, 0)),
            pl.BlockSpec((1, D), lambda b, c: (0, 0)),
        ],
        out_specs=[
            pl.BlockSpec((1, k, 1), lambda b, c: (b, 0, 0)),
            pl.BlockSpec((1, k, 1), lambda b, c: (b, 0, 0)),
            pl.BlockSpec((1, nb, 1), lambda b, c: (b, 0, 0)),
        ],
        out_shape=[
            jax.ShapeDtypeStruct((B, k, 1), jnp.int32),
            jax.ShapeDtypeStruct((B, k, 1), jnp.float32),
            jax.ShapeDtypeStruct((B, nb, 1), jnp.int32),
        ],
        scratch_shapes=[pltpu.VMEM((1, T), jnp.float32)],
    )(hidden_states, w_router.reshape(1, D))
    return gidx, gate, ends


def _gather(hs2d, gidx_flat):
    n = gidx_flat.shape[0]
    d = hs2d.shape[1]
    info = plsc.get_sparse_core_info()
    nw = info.num_cores * info.num_subcores
    rpw = n // nw
    rch = min(rpw, 32)
    mesh = plsc.VectorSubcoreMesh(core_axis_name="c", subcore_axis_name="s")

    @functools.partial(
        pl.kernel,
        out_type=jax.ShapeDtypeStruct((n, d), jnp.float32),
        mesh=mesh,
        scratch_types=[
            pltpu.VMEM((rch,), jnp.int32),
            pltpu.VMEM((rch, d), jnp.float32),
            pltpu.SemaphoreType.DMA,
        ],
    )
    def gather_kernel(hs_ref, gidx_ref, out_ref, idx_v, rows_v, sem):
        wid = lax.axis_index("s") * info.num_cores + lax.axis_index("c")
        base = wid * rpw
        for h in range(rpw // rch):
            off = base + h * rch
            pltpu.sync_copy(gidx_ref.at[pl.ds(off, rch)], idx_v)
            pltpu.async_copy(hs_ref.at[idx_v], rows_v, sem).wait()
            pltpu.sync_copy(rows_v, out_ref.at[pl.ds(off, rch)])

    return gather_kernel(hs2d, gidx_flat)


def _qkv_body(x_ref, wq_ref, wk_ref, wv_ref, bq_ref, bk_ref, bv_ref,
              cos_ref, sin_ref, ln1_ref, q_ref, k_ref, v_ref):
    x = x_ref[...]
    var = jnp.mean(x * x, axis=-1, keepdims=True)
    h = (x * lax.rsqrt(var + _EPS) * ln1_ref[...]).astype(jnp.bfloat16)
    wq = wq_ref[...].astype(jnp.bfloat16)
    wk = wk_ref[...].astype(jnp.bfloat16)
    wv = wv_ref[...].astype(jnp.bfloat16)
    q = jnp.dot(h, wq, preferred_element_type=jnp.float32) + bq_ref[...]
    kk = jnp.dot(h, wk, preferred_element_type=jnp.float32) + bk_ref[...]
    v = jnp.dot(h, wv, preferred_element_type=jnp.float32) + bv_ref[...]
    cos = cos_ref[...]
    sin = sin_ref[...]

    def rope(z):
        outs = []
        for hh in range(z.shape[1] // _HD):
            zz = z[:, hh * _HD:(hh + 1) * _HD]
            z1 = zz[:, :_HD // 2]
            z2 = zz[:, _HD // 2:]
            rot = jnp.concatenate([-z2, z1], axis=1)
            outs.append(zz * cos + rot * sin)
        return jnp.concatenate(outs, axis=1)

    q_ref[...] = rope(q).astype(jnp.bfloat16)
    k_ref[...] = rope(kk).astype(jnp.bfloat16)
    v_ref[...] = v.astype(jnp.bfloat16)


def _qkv(sel, wq, bq, wk, bk, wv, bv, ln1_w, cos, sin):
    n, d = sel.shape
    hhd = wq.shape[1]
    rb, cb = 512, 512
    outs = pl.pallas_call(
        _qkv_body,
        grid=(n // rb, hhd // cb),
        in_specs=[
            pl.BlockSpec((rb, d), lambda r, c: (r, 0)),
            pl.BlockSpec((d, cb), lambda r, c: (0, c)),
            pl.BlockSpec((d, cb), lambda r, c: (0, c)),
            pl.BlockSpec((d, cb), lambda r, c: (0, c)),
            pl.BlockSpec((1, cb), lambda r, c: (0, c)),
            pl.BlockSpec((1, cb), lambda r, c: (0, c)),
            pl.BlockSpec((1, cb), lambda r, c: (0, c)),
            pl.BlockSpec((rb, _HD), lambda r, c: (r, 0)),
            pl.BlockSpec((rb, _HD), lambda r, c: (r, 0)),
            pl.BlockSpec((1, d), lambda r, c: (0, 0)),
        ],
        out_specs=[pl.BlockSpec((rb, cb), lambda r, c: (r, c))] * 3,
        out_shape=[jax.ShapeDtypeStruct((n, hhd), jnp.bfloat16)] * 3,
    )(sel, wq, wk, wv, bq.reshape(1, hhd), bk.reshape(1, hhd),
      bv.reshape(1, hhd), cos, sin, ln1_w.reshape(1, d))
    return outs


def _attn_body(q_ref, k_ref, v_ref, o_ref, *, bq, scale):
    qi = pl.program_id(1)
    q = q_ref[...]
    kd = k_ref[pl.ds(qi * bq, bq), :]
    vd = v_ref[pl.ds(qi * bq, bq), :]
    s = lax.dot_general(q, kd, (((1,), (1,)), ((), ())),
                        preferred_element_type=jnp.float32) * scale
    rows = lax.broadcasted_iota(jnp.int32, s.shape, 0)
    cols = lax.broadcasted_iota(jnp.int32, s.shape, 1)
    s = jnp.where(cols <= rows, s, jnp.float32(-1e9))
    m0 = jnp.max(s, axis=-1, keepdims=True)
    p0 = jnp.exp(s - m0)
    l0 = jnp.sum(p0, axis=-1, keepdims=True)
    acc0 = jnp.dot(p0.astype(jnp.bfloat16), vd, preferred_element_type=jnp.float32)

    def chunk(c, carry):
        m_run, l_run, acc = carry
        kb = k_ref[pl.ds(c * bq, bq), :]
        vb = v_ref[pl.ds(c * bq, bq), :]
        sc = lax.dot_general(q, kb, (((1,), (1,)), ((), ())),
                             preferred_element_type=jnp.float32) * scale
        m_new = jnp.maximum(m_run, jnp.max(sc, axis=-1, keepdims=True))
        p = jnp.exp(sc - m_new)
        corr = jnp.exp(m_run - m_new)
        l_new = l_run * corr + jnp.sum(p, axis=-1, keepdims=True)
        acc_new = acc * corr + jnp.dot(p.astype(jnp.bfloat16), vb,
                                       preferred_element_type=jnp.float32)
        return m_new, l_new, acc_new

    m, l, acc = lax.fori_loop(0, qi, chunk, (m0, l0, acc0))
    o_ref[...] = (acc / l).astype(jnp.bfloat16)


def _attn(q, k, v):
    n, hhd = q.shape
    nh = hhd // _HD
    bq = 512
    return pl.pallas_call(
        functools.partial(_attn_body, bq=bq, scale=1.0 / float(np.sqrt(_HD))),
        grid=(nh, n // bq),
        in_specs=[
            pl.BlockSpec((bq, _HD), lambda h, qi: (qi, h)),
            pl.BlockSpec((n, _HD), lambda h, qi: (0, h)),
            pl.BlockSpec((n, _HD), lambda h, qi: (0, h)),
        ],
        out_specs=pl.BlockSpec((bq, _HD), lambda h, qi: (qi, h)),
        out_shape=jax.ShapeDtypeStruct((n, hhd), jnp.bfloat16),
    )(q, k, v)


def _oproj_body(att_ref, wo_ref, sel_ref, ln2_ref, x1_ref, h2_ref):
    x1 = jnp.dot(att_ref[...], wo_ref[...].astype(jnp.bfloat16),
                 preferred_element_type=jnp.float32) + sel_ref[...]
    x1_ref[...] = x1
    var = jnp.mean(x1 * x1, axis=-1, keepdims=True)
    h2_ref[...] = (x1 * lax.rsqrt(var + _EPS) * ln2_ref[...]).astype(jnp.bfloat16)


def _oproj(att, wo, sel, ln2_w):
    n, d = sel.shape
    rb = 512
    return pl.pallas_call(
        _oproj_body,
        grid=(n // rb,),
        in_specs=[
            pl.BlockSpec((rb, wo.shape[0]), lambda r: (r, 0)),
            pl.BlockSpec(wo.shape, lambda r: (0, 0)),
            pl.BlockSpec((rb, d), lambda r: (r, 0)),
            pl.BlockSpec((1, d), lambda r: (0, 0)),
        ],
        out_specs=[pl.BlockSpec((rb, d), lambda r: (r, 0))] * 2,
        out_shape=[jax.ShapeDtypeStruct((n, d), jnp.float32),
                   jax.ShapeDtypeStruct((n, d), jnp.bfloat16)],
    )(att, wo, sel, ln2_w.reshape(1, d))


def _mlp_body(h2_ref, wg_ref, wu_ref, wd_ref, out_ref):
    i = pl.program_id(0)
    h2 = h2_ref[...]
    g = jnp.dot(h2, wg_ref[...].astype(jnp.bfloat16),
                preferred_element_type=jnp.float32)
    u = jnp.dot(h2, wu_ref[...].astype(jnp.bfloat16),
                preferred_element_type=jnp.float32)
    m = (g * (1.0 / (1.0 + jnp.exp(-g))) * u).astype(jnp.bfloat16)
    part = jnp.dot(m, wd_ref[...].astype(jnp.bfloat16),
                   preferred_element_type=jnp.float32)

    @pl.when(i == 0)
    def _():
        out_ref[...] = part

    @pl.when(i > 0)
    def _():
        out_ref[...] += part


def _mlp(h2, w_gate, w_up, w_down):
    n, d = h2.shape
    i_dim = w_gate.shape[1]
    ci = 256
    return pl.pallas_call(
        _mlp_body,
        grid=(i_dim // ci,),
        in_specs=[
            pl.BlockSpec((n, d), lambda i: (0, 0)),
            pl.BlockSpec((d, ci), lambda i: (0, i)),
            pl.BlockSpec((d, ci), lambda i: (0, i)),
            pl.BlockSpec((ci, d), lambda i: (i, 0)),
        ],
        out_specs=pl.BlockSpec((n, d), lambda i: (0, 0)),
        out_shape=jax.ShapeDtypeStruct((n, d), jnp.float32),
    )(h2, w_gate, w_up, w_down)


def _blend_body(mlp_ref, x1_ref, sel_ref, gate_ref, out_ref):
    y = mlp_ref[...] + x1_ref[...]
    gt = gate_ref[...]
    out_ref[...] = gt * y + (1.0 - gt) * sel_ref[...]


def _blend(mlp_out, x1, sel, gate_col):
    n, d = x1.shape
    rb = 512
    return pl.pallas_call(
        _blend_body,
        grid=(n // rb,),
        in_specs=[
            pl.BlockSpec((rb, d), lambda r: (r, 0)),
            pl.BlockSpec((rb, d), lambda r: (r, 0)),
            pl.BlockSpec((rb, d), lambda r: (r, 0)),
            pl.BlockSpec((rb, 1), lambda r: (r, 0)),
        ],
        out_specs=pl.BlockSpec((rb, d), lambda r: (r, 0)),
        out_shape=jax.ShapeDtypeStruct((n, d), jnp.float32),
    )(mlp_out, x1, sel, gate_col)


def _merge_body(starts_sref, gidx_sref, hs_ref, gated_ref, out_ref, *, rb):
    b = pl.program_id(0)
    j = pl.program_id(1)
    nj = pl.num_programs(1)
    out_ref[...] = hs_ref[...]
    base = b * (nj * rb) + j * rb

    def patch(n, carry):
        local = gidx_sref[n] - base
        out_ref[0, pl.ds(local, 1), :] = gated_ref[pl.ds(n, 1), :]
        return carry

    m = b * nj + j
    lax.fori_loop(starts_sref[m], starts_sref[m + 1], patch, 0)


def _merge(starts, gidx_flat, hidden_states, gated):
    B, T, D = hidden_states.shape
    n = gated.shape[0]
    rb = 512
    grid_spec = pltpu.PrefetchScalarGridSpec(
        num_scalar_prefetch=2,
        grid=(B, T // rb),
        in_specs=[
            pl.BlockSpec((1, rb, D), lambda b, j, s, g: (b, j, 0)),
            pl.BlockSpec((n, D), lambda b, j, s, g: (0, 0)),
        ],
        out_specs=pl.BlockSpec((1, rb, D), lambda b, j, s, g: (b, j, 0)),
    )
    return pl.pallas_call(
        functools.partial(_merge_body, rb=rb),
        grid_spec=grid_spec,
        out_shape=jax.ShapeDtypeStruct((B, T, D), jnp.float32),
    )(starts, gidx_flat, hidden_states, gated)


def kernel(hidden_states, w_router, wq, bq, wk, bk, wv, bv, wo,
           w_gate, w_up, w_down, ln1_w, ln2_w):
    B, T, D = hidden_states.shape
    k = max(1, int(T * _CAPACITY))
    n = B * k

    gidx, gate, ends = _route(hidden_states, w_router, k)
    gidx_flat = gidx.reshape(n)
    gate_col = gate.reshape(n, 1)
    hs2d = hidden_states.reshape(B * T, D)
    nb = T // 512
    ends_flat = ends.reshape(B, nb) + (jnp.arange(B, dtype=jnp.int32) * k)[:, None]
    starts = jnp.concatenate(
        [jnp.zeros((1,), jnp.int32), ends_flat.reshape(B * nb)])

    sel = _gather(hs2d, gidx_flat)

    pos = jnp.arange(n, dtype=jnp.float32)
    inv = 1.0 / (10000.0 ** (jnp.arange(0, _HD, 2, dtype=jnp.float32) / _HD))
    ang = pos[:, None] * inv[None, :]
    cos = jnp.concatenate([jnp.cos(ang), jnp.cos(ang)], axis=-1)
    sin = jnp.concatenate([jnp.sin(ang), jnp.sin(ang)], axis=-1)

    q, kk, v = _qkv(sel, wq, bq, wk, bk, wv, bv, ln1_w, cos, sin)
    att = _attn(q, kk, v)
    x1, h2 = _oproj(att, wo, sel, ln2_w)
    mlp_out = _mlp(h2, w_gate, w_up, w_down)
    gated = _blend(mlp_out, x1, sel, gate_col)

    return _merge(starts, gidx_flat, hidden_states, gated)
